# Initial kernel scaffold; baseline (speedup 1.0000x reference)
import jax
import jax.numpy as jnp
from jax.experimental import pallas as pl


def kernel(hidden_state, key_seq, value_matrix, key_mask_matrix, key_table, val_table):
    raise NotImplementedError("write your pallas kernel here")


# speedup vs baseline: 1.1652x; 1.1652x over previous
import jax
import jax.numpy as jnp
from jax.experimental import pallas as pl
from jax.experimental.pallas import tpu as pltpu

QB = 64
U = 16


def _tamm_kernel(hid_ref, ks_hbm, vm_hbm, mask_ref, kt_ref, vt_ref, out_ref,
                 ek_ref, res_ref, p_vmem, ks_smem, vm_smem, p_smem,
                 sem_ks, sem_vm, sem_p):
    b = pl.program_id(0)
    qb = pl.program_id(1)
    lk = ek_ref.shape[0]
    h = ek_ref.shape[1]
    temper = h ** 0.5

    cp_vm = pltpu.make_async_copy(
        vm_hbm.at[b, pl.ds(qb * QB, QB), :], vm_smem, sem_vm)
    cp_vm.start()

    @pl.when(qb == 0)
    def _():
        cp_ks = pltpu.make_async_copy(ks_hbm.at[b, :], ks_smem, sem_ks)
        cp_ks.start()
        cp_ks.wait()
        for k0 in range(0, lk, 8):
            rows = [kt_ref[ks_smem[k0 + j]] for j in range(8)]
            ek_ref[k0:k0 + 8, :] = jnp.concatenate(rows, axis=0)

    hid = hid_ref[0]
    ek = ek_ref[...]
    u = jax.lax.dot_general(hid, ek, (((1,), (1,)), ((), ())),
                            preferred_element_type=jnp.float32) / temper
    mask = jnp.clip(mask_ref[0], 0, 1).astype(jnp.float32)
    delta = jnp.exp(u) * mask
    denom = jnp.sum(delta, axis=1, keepdims=True) + 1e-10
    p = delta / denom
    p_vmem[...] = p
    ko = jax.lax.dot_general(p, ek, (((1,), (0,)), ((), ())),
                             preferred_element_type=jnp.float32)
    res_ref[...] = ko + hid

    cp_p = pltpu.make_async_copy(p_vmem, p_smem, sem_p)
    cp_p.start()
    cp_p.wait()
    cp_vm.wait()

    zrow = jnp.zeros((1, h), jnp.float32)

    for qi in range(QB):
        def body(c, accs, qi=qi):
            a0, a1, a2, a3 = accs
            base = c * U
            rows = []
            for j in range(U):
                idx = vm_smem[qi, base + j]
                w = p_smem[qi, base + j]
                rows.append(vt_ref[idx] * w)
            for j in range(0, U, 4):
                a0 = a0 + rows[j]
                a1 = a1 + rows[j + 1]
                a2 = a2 + rows[j + 2]
                a3 = a3 + rows[j + 3]
            return (a0, a1, a2, a3)

        a0, a1, a2, a3 = jax.lax.fori_loop(0, lk // U, body,
                                           (zrow, zrow, zrow, zrow))
        out_ref[qi] = (a0 + a1) + (a2 + a3) + res_ref[qi:qi + 1, :]


def kernel(hidden_state, key_seq, value_matrix, key_mask_matrix,
           key_table, val_table):
    b, lq, h = hidden_state.shape
    lk = key_seq.shape[1]
    ks_rows = key_table.shape[0]
    vs_rows = val_table.shape[0]
    kt3 = key_table.reshape(ks_rows, 1, h)
    vt3 = val_table.reshape(vs_rows, 1, h)
    nq = lq // QB

    out = pl.pallas_call(
        _tamm_kernel,
        out_shape=jax.ShapeDtypeStruct((b * lq, 1, h), jnp.float32),
        grid=(b, nq),
        in_specs=[
            pl.BlockSpec((1, QB, h), lambda i, j: (i, j, 0)),
            pl.BlockSpec(memory_space=pl.ANY),
            pl.BlockSpec(memory_space=pl.ANY),
            pl.BlockSpec((1, QB, lk), lambda i, j: (i, j, 0)),
            pl.BlockSpec((ks_rows, 1, h), lambda i, j: (0, 0, 0)),
            pl.BlockSpec((vs_rows, 1, h), lambda i, j: (0, 0, 0)),
        ],
        out_specs=pl.BlockSpec((QB, 1, h),
                               lambda i, j: (i * (lq // QB) + j, 0, 0)),
        scratch_shapes=[
            pltpu.VMEM((lk, h), jnp.float32),
            pltpu.VMEM((QB, h), jnp.float32),
            pltpu.VMEM((QB, lk), jnp.float32),
            pltpu.SMEM((lk,), jnp.int32),
            pltpu.SMEM((QB, lk), jnp.int32),
            pltpu.SMEM((QB, lk), jnp.float32),
            pltpu.SemaphoreType.DMA,
            pltpu.SemaphoreType.DMA,
            pltpu.SemaphoreType.DMA,
        ],
        compiler_params=pltpu.CompilerParams(
            dimension_semantics=("parallel", "arbitrary"),
            vmem_limit_bytes=48 * 1024 * 1024,
        ),
        name="tamm_fused",
    )(hidden_state, key_seq, value_matrix, key_mask_matrix, kt3, vt3)
    return out.reshape(b, lq, h)


# speedup vs baseline: 2.9649x; 2.5445x over previous
import jax
import jax.numpy as jnp
from jax.experimental import pallas as pl
from jax.experimental.pallas import tpu as pltpu

QB = 64


def _tamm_kernel(hid_ref, ks_hbm, vm_hbm, mask_ref, kt_ref, vt_ref, out_ref,
                 ek_ref, res3_ref, p3_ref, tile_a, tile_b, ks_smem, vm_smem,
                 sem_ks, sem_vm):
    b = pl.program_id(0)
    qb = pl.program_id(1)
    lk = ek_ref.shape[0]
    h = ek_ref.shape[1]
    temper = h ** 0.5

    cp_vm = pltpu.make_async_copy(
        vm_hbm.at[b, pl.ds(qb * QB, QB), :], vm_smem, sem_vm)
    cp_vm.start()

    @pl.when(qb == 0)
    def _():
        cp_ks = pltpu.make_async_copy(ks_hbm.at[b, :], ks_smem, sem_ks)
        cp_ks.start()
        cp_ks.wait()
        for k0 in range(0, lk, 8):
            rows = [kt_ref[ks_smem[k0 + j]] for j in range(8)]
            ek_ref[k0:k0 + 8, :] = jnp.concatenate(rows, axis=0)

    hid = hid_ref[0]
    ek = ek_ref[...]
    u = jax.lax.dot_general(hid, ek, (((1,), (1,)), ((), ())),
                            preferred_element_type=jnp.float32) / temper
    mask = jnp.clip(mask_ref[0], 0, 1).astype(jnp.float32)
    delta = jnp.exp(u) * mask
    denom = jnp.sum(delta, axis=1, keepdims=True) + 1e-10
    p = delta / denom
    p3_ref[...] = p.reshape(QB, 1, lk)
    ko = jax.lax.dot_general(p, ek, (((1,), (0,)), ((), ())),
                             preferred_element_type=jnp.float32)
    res3_ref[...] = (ko + hid).reshape(QB, 1, h)
    cp_vm.wait()

    def body(t, carry, ):
        for tile in (tile_a, tile_b):
            qi = t * 2 + (0 if tile is tile_a else 1)
            for k0 in range(0, lk, 8):
                rows = [vt_ref[vm_smem[qi, k0 + j]] for j in range(8)]
                tile[k0:k0 + 8, :] = jnp.concatenate(rows, axis=0)
            prow = p3_ref[qi]
            o = jax.lax.dot_general(prow, tile[...], (((1,), (0,)), ((), ())),
                                    preferred_element_type=jnp.float32)
            out_ref[qi] = o + res3_ref[qi]
        return carry

    jax.lax.fori_loop(0, QB // 2, body, 0)


def kernel(hidden_state, key_seq, value_matrix, key_mask_matrix,
           key_table, val_table):
    b, lq, h = hidden_state.shape
    lk = key_seq.shape[1]
    ks_rows = key_table.shape[0]
    vs_rows = val_table.shape[0]
    kt3 = key_table.reshape(ks_rows, 1, h)
    vt3 = val_table.reshape(vs_rows, 1, h)
    nq = lq // QB

    out = pl.pallas_call(
        _tamm_kernel,
        out_shape=jax.ShapeDtypeStruct((b * lq, 1, h), jnp.float32),
        grid=(b, nq),
        in_specs=[
            pl.BlockSpec((1, QB, h), lambda i, j: (i, j, 0)),
            pl.BlockSpec(memory_space=pl.ANY),
            pl.BlockSpec(memory_space=pl.ANY),
            pl.BlockSpec((1, QB, lk), lambda i, j: (i, j, 0)),
            pl.BlockSpec((ks_rows, 1, h), lambda i, j: (0, 0, 0)),
            pl.BlockSpec((vs_rows, 1, h), lambda i, j: (0, 0, 0)),
        ],
        out_specs=pl.BlockSpec((QB, 1, h),
                               lambda i, j: (i * (lq // QB) + j, 0, 0)),
        scratch_shapes=[
            pltpu.VMEM((lk, h), jnp.float32),
            pltpu.VMEM((QB, 1, h), jnp.float32),
            pltpu.VMEM((QB, 1, lk), jnp.float32),
            pltpu.VMEM((lk, h), jnp.float32),
            pltpu.VMEM((lk, h), jnp.float32),
            pltpu.SMEM((lk,), jnp.int32),
            pltpu.SMEM((QB, lk), jnp.int32),
            pltpu.SemaphoreType.DMA,
            pltpu.SemaphoreType.DMA,
        ],
        compiler_params=pltpu.CompilerParams(
            dimension_semantics=("parallel", "arbitrary"),
            vmem_limit_bytes=48 * 1024 * 1024,
        ),
        name="tamm_fused",
    )(hidden_state, key_seq, value_matrix, key_mask_matrix, kt3, vt3)
    return out.reshape(b, lq, h)
